# baseline (speedup 1.0000x reference)
import functools

import jax
import jax.numpy as jnp
from jax.experimental import pallas as pl
from jax.experimental.pallas import tpu as pltpu

HEADS = 4
TOPK = 3
NEG_SLOPE = 0.2


def _rownorm(v):
    n = jnp.sqrt(jnp.sum(v * v, axis=-1, keepdims=True))
    return v / jnp.maximum(n, 1e-8)


def _topk_mask(sim, L):
    cols = [sim[:, j:j + 1] for j in range(L)]
    outs = []
    for j in range(L):
        r = jnp.zeros_like(cols[0])
        for k in range(L):
            if k == j:
                continue
            if k < j:
                beat = cols[k] >= cols[j]
            else:
                beat = cols[k] > cols[j]
            r = r + beat.astype(jnp.float32)
        outs.append((r < (TOPK - 0.5)).astype(jnp.float32))
    return jnp.concatenate(outs, axis=1)


def _edges_kernel(t_ref, l_ref, i_ref, a_ref, *, T, L, I):
    t = t_ref[0]
    lab = l_ref[0]
    img = i_ref[0]
    tn = _rownorm(t)
    labn = _rownorm(lab)
    imgn = _rownorm(img)
    dn = (((1,), (1,)), ((), ()))
    sim_l = jax.lax.dot_general(tn, labn, dn, preferred_element_type=jnp.float32)
    sim_i = jax.lax.dot_general(tn, imgn, dn, preferred_element_type=jnp.float32)
    mask_l = _topk_mask(sim_l, L)
    mask_i = _topk_mask(sim_i, I)
    eye_l = (jax.lax.broadcasted_iota(jnp.int32, (L, L), 0)
             == jax.lax.broadcasted_iota(jnp.int32, (L, L), 1)).astype(jnp.float32)
    mask_lT = jax.lax.dot_general(eye_l, mask_l, dn,
                                  preferred_element_type=jnp.float32)
    eye_i = (jax.lax.broadcasted_iota(jnp.int32, (I, I), 0)
             == jax.lax.broadcasted_iota(jnp.int32, (I, I), 1)).astype(jnp.float32)
    mask_iT = jax.lax.dot_general(eye_i, mask_i, dn,
                                  preferred_element_type=jnp.float32)

    r = jax.lax.broadcasted_iota(jnp.int32, (T, T), 0)
    c = jax.lax.broadcasted_iota(jnp.int32, (T, T), 1)
    chain = ((r - c == 1) | (c - r == 1) | (r == c)).astype(jnp.float32)

    ones_ll = jnp.ones((L, L), jnp.float32)
    ones_ii = jnp.ones((I, I), jnp.float32)
    ones_il = jnp.ones((I, L), jnp.float32)
    ones_li = jnp.ones((L, I), jnp.float32)

    a_ref[0, 0:T, 0:T] = chain
    a_ref[0, 0:T, T:T + L] = mask_l
    a_ref[0, 0:T, T + L:T + L + I] = mask_i
    a_ref[0, T:T + L, 0:T] = mask_lT
    a_ref[0, T:T + L, T:T + L] = ones_ll + eye_l
    a_ref[0, T:T + L, T + L:T + L + I] = ones_li
    a_ref[0, T + L:T + L + I, 0:T] = mask_iT
    a_ref[0, T + L:T + L + I, T:T + L] = ones_il
    a_ref[0, T + L:T + L + I, T + L:T + L + I] = ones_ii + eye_i


def _layer_kernel(x_ref, a_ref, w_ref, as_ref, ad_ref, b_ref, g_ref, be_ref,
                  o_ref, *, N, H):
    out_ch = H // HEADS
    x = x_ref[0]
    A = a_ref[0]
    h = jnp.dot(x, w_ref[...], preferred_element_type=jnp.float32)
    dn = (((1,), (1,)), ((), ()))
    aggs = []
    for hd in range(HEADS):
        hh = h[:, hd * out_ch:(hd + 1) * out_ch]
        asr = as_ref[hd:hd + 1, :]
        adr = ad_ref[hd:hd + 1, :]
        a_src = jax.lax.dot_general(asr, hh, dn,
                                    preferred_element_type=jnp.float32)
        a_dst = jax.lax.dot_general(hh, adr, dn,
                                    preferred_element_type=jnp.float32)
        alpha = a_dst + a_src
        alpha = jnp.where(alpha >= 0, alpha, NEG_SLOPE * alpha)
        malpha = jnp.where(A > 0, alpha, -1e30)
        amax = jnp.max(malpha, axis=1, keepdims=True)
        ex = A * jnp.exp(jnp.minimum(alpha - amax, 0.0))
        den = jnp.sum(ex, axis=1, keepdims=True)
        P = ex / (den + 1e-16)
        aggs.append(jnp.dot(P, hh, preferred_element_type=jnp.float32))
    agg = jnp.concatenate(aggs, axis=1)
    out = jnp.maximum(agg + b_ref[...], 0.0)
    y = out + x
    mu = jnp.mean(y, axis=1, keepdims=True)
    yc = y - mu
    var = jnp.mean(yc * yc, axis=1, keepdims=True)
    o_ref[0] = yc / jnp.sqrt(var + 1e-5) * g_ref[...] + be_ref[...]


def _build_A(text, label, image, interpret=False):
    B, T, H = text.shape
    L = label.shape[1]
    I = image.shape[1]
    N = T + L + I
    return pl.pallas_call(
        functools.partial(_edges_kernel, T=T, L=L, I=I),
        grid=(B,),
        in_specs=[
            pl.BlockSpec((1, T, H), lambda b: (b, 0, 0)),
            pl.BlockSpec((1, L, H), lambda b: (b, 0, 0)),
            pl.BlockSpec((1, I, H), lambda b: (b, 0, 0)),
        ],
        out_specs=pl.BlockSpec((1, N, N), lambda b: (b, 0, 0)),
        out_shape=jax.ShapeDtypeStruct((B, N, N), jnp.float32),
        interpret=interpret,
    )(text, label, image)


def _layer(x, A, W, a_s, a_d, b, g, be, interpret=False):
    B, N, H = x.shape
    out_ch = H // HEADS
    return pl.pallas_call(
        functools.partial(_layer_kernel, N=N, H=H),
        grid=(B,),
        in_specs=[
            pl.BlockSpec((1, N, H), lambda b: (b, 0, 0)),
            pl.BlockSpec((1, N, N), lambda b: (b, 0, 0)),
            pl.BlockSpec((H, H), lambda b: (0, 0)),
            pl.BlockSpec((HEADS, out_ch), lambda b: (0, 0)),
            pl.BlockSpec((HEADS, out_ch), lambda b: (0, 0)),
            pl.BlockSpec((1, H), lambda b: (0, 0)),
            pl.BlockSpec((1, H), lambda b: (0, 0)),
            pl.BlockSpec((1, H), lambda b: (0, 0)),
        ],
        out_specs=pl.BlockSpec((1, N, H), lambda b: (b, 0, 0)),
        out_shape=jax.ShapeDtypeStruct((B, N, H), jnp.float32),
        interpret=interpret,
    )(x, A, W, a_s, a_d, b, g, be)


def _edge_mask(t, lab, img, T, L, I):
    tn = _rownorm(t)
    labn = _rownorm(lab)
    imgn = _rownorm(img)
    dn = (((1,), (1,)), ((), ()))
    sim_l = jax.lax.dot_general(tn, labn, dn, preferred_element_type=jnp.float32)
    sim_i = jax.lax.dot_general(tn, imgn, dn, preferred_element_type=jnp.float32)
    mask_l = _topk_mask(sim_l, L)
    mask_i = _topk_mask(sim_i, I)
    eye_l = (jax.lax.broadcasted_iota(jnp.int32, (L, L), 0)
             == jax.lax.broadcasted_iota(jnp.int32, (L, L), 1)).astype(jnp.float32)
    mask_lT = jax.lax.dot_general(eye_l, mask_l, dn,
                                  preferred_element_type=jnp.float32)
    eye_i = (jax.lax.broadcasted_iota(jnp.int32, (I, I), 0)
             == jax.lax.broadcasted_iota(jnp.int32, (I, I), 1)).astype(jnp.float32)
    mask_iT = jax.lax.dot_general(eye_i, mask_i, dn,
                                  preferred_element_type=jnp.float32)
    r = jax.lax.broadcasted_iota(jnp.int32, (T, T), 0)
    c = jax.lax.broadcasted_iota(jnp.int32, (T, T), 1)
    chain = ((r - c == 1) | (c - r == 1) | (r == c)).astype(jnp.float32)
    return chain, mask_l, mask_i, mask_lT, mask_iT, eye_l, eye_i


def _gat_layer(x, A, W, a_src_w, a_dst_w, b, g, be, N, H):
    out_ch = H // HEADS
    h = jnp.dot(x, W, preferred_element_type=jnp.float32)
    dn = (((1,), (1,)), ((), ()))
    aggs = []
    for hd in range(HEADS):
        hh = h[:, hd * out_ch:(hd + 1) * out_ch]
        asr = a_src_w[hd:hd + 1, :]
        adr = a_dst_w[hd:hd + 1, :]
        a_s = jax.lax.dot_general(asr, hh, dn,
                                  preferred_element_type=jnp.float32)
        a_d = jax.lax.dot_general(hh, adr, dn,
                                  preferred_element_type=jnp.float32)
        alpha = a_d + a_s
        alpha = jnp.where(alpha >= 0, alpha, NEG_SLOPE * alpha)
        malpha = jnp.where(A > 0, alpha, -1e30)
        amax = jnp.max(malpha, axis=1, keepdims=True)
        ex = A * jnp.exp(jnp.minimum(alpha - amax, 0.0))
        den = jnp.sum(ex, axis=1, keepdims=True)
        P = ex / (den + 1e-16)
        aggs.append(jnp.dot(P, hh, preferred_element_type=jnp.float32))
    agg = jnp.concatenate(aggs, axis=1)
    out = jnp.maximum(agg + b, 0.0)
    y = out + x
    mu = jnp.mean(y, axis=1, keepdims=True)
    yc = y - mu
    var = jnp.mean(yc * yc, axis=1, keepdims=True)
    return yc / jnp.sqrt(var + 1e-5) * g + be


def _gat_layer_band(x, mask_text, rowmask_li, W, a_src_w, a_dst_w, b, g, be,
                    T, N, H):
    out_ch = H // HEADS
    h = jnp.dot(x, W, preferred_element_type=jnp.float32)
    dn = (((1,), (1,)), ((), ()))
    aggs = []
    for hd in range(HEADS):
        hh = h[:, hd * out_ch:(hd + 1) * out_ch]
        asr = a_src_w[hd:hd + 1, :]
        adr = a_dst_w[hd:hd + 1, :]
        a_s_col = jax.lax.dot_general(hh, asr, dn,
                                      preferred_element_type=jnp.float32)
        a_d_col = jax.lax.dot_general(hh, adr, dn,
                                      preferred_element_type=jnp.float32)
        s_0 = a_s_col[0:T]
        s_m1 = jnp.concatenate([a_s_col[0:1], a_s_col[0:T - 1]], axis=0)
        s_p1 = a_s_col[1:T + 1]
        a_s_li = jax.lax.dot_general(asr, hh[T:N], dn,
                                     preferred_element_type=jnp.float32)
        alpha_t = jnp.concatenate(
            [s_m1, s_0, s_p1, jnp.broadcast_to(a_s_li, (T, N - T))], axis=1)
        alpha_t = a_d_col[0:T] + alpha_t
        alpha_t = jnp.where(alpha_t >= 0, alpha_t, NEG_SLOPE * alpha_t)
        amax_t = jnp.max(jnp.where(mask_text > 0, alpha_t, -1e30),
                         axis=1, keepdims=True)
        ex_t = mask_text * jnp.exp(jnp.minimum(alpha_t - amax_t, 0.0))
        den_t = jnp.sum(ex_t, axis=1, keepdims=True)
        P_t = ex_t / (den_t + 1e-16)
        h_m1 = jnp.concatenate([hh[0:1], hh[0:T - 1]], axis=0)
        h_p1 = hh[1:T + 1]
        agg_t = (P_t[:, 0:1] * h_m1 + P_t[:, 1:2] * hh[0:T]
                 + P_t[:, 2:3] * h_p1
                 + jnp.dot(P_t[:, 3:3 + (N - T)], hh[T:N],
                           preferred_element_type=jnp.float32))
        a_s_row = jax.lax.dot_general(asr, hh, dn,
                                      preferred_element_type=jnp.float32)
        alpha_r = a_d_col[T:N] + a_s_row
        alpha_r = jnp.where(alpha_r >= 0, alpha_r, NEG_SLOPE * alpha_r)
        amax_r = jnp.max(jnp.where(rowmask_li > 0, alpha_r, -1e30),
                         axis=1, keepdims=True)
        ex_r = rowmask_li * jnp.exp(jnp.minimum(alpha_r - amax_r, 0.0))
        den_r = jnp.sum(ex_r, axis=1, keepdims=True)
        P_r = ex_r / (den_r + 1e-16)
        agg_r = jnp.dot(P_r, hh, preferred_element_type=jnp.float32)
        aggs.append(jnp.concatenate([agg_t, agg_r], axis=0))
    agg = jnp.concatenate(aggs, axis=1)
    out = jnp.maximum(agg + b, 0.0)
    y = out + x
    mu = jnp.mean(y, axis=1, keepdims=True)
    yc = y - mu
    var = jnp.mean(yc * yc, axis=1, keepdims=True)
    return yc / jnp.sqrt(var + 1e-5) * g + be


def _gat_layer_band2(x, mask_text, colmask_li, Wext, b, g, be, li_scr,
                     T, N, H):
    out_ch = H // HEADS
    hext = jnp.dot(x, Wext, preferred_element_type=jnp.float32)
    h = hext[:, 0:H]
    asd = hext[:, H:H + 2 * HEADS]
    eye8 = (jax.lax.broadcasted_iota(jnp.int32, (2 * HEADS, 2 * HEADS), 0)
            == jax.lax.broadcasted_iota(jnp.int32,
                                        (2 * HEADS, 2 * HEADS), 1)
            ).astype(jnp.float32)
    asd_liT = jax.lax.dot_general(asd[T:N], eye8, (((0,), (0,)), ((), ())),
                                  preferred_element_type=jnp.float32)
    h_dn = jnp.concatenate([h[0:1], h[0:T - 1]], axis=0)
    h_up = h[1:T + 1]
    li_scr[...] = jnp.zeros((HEADS * (N - T), H), jnp.float32)
    for hd in range(HEADS):
        c0 = hd * out_ch
        li_scr[hd * (N - T):(hd + 1) * (N - T), c0:c0 + out_ch] = \
            h[T:N, c0:c0 + out_ch]
    band_parts = []
    pli_parts = []
    aggr_parts = []
    for hd in range(HEADS):
        c0 = hd * out_ch
        a_s_col = asd[:, hd:hd + 1]
        a_d_col = asd[:, HEADS + hd:HEADS + hd + 1]
        s_0 = a_s_col[0:T]
        s_m1 = jnp.concatenate([a_s_col[0:1], a_s_col[0:T - 1]], axis=0)
        s_p1 = a_s_col[1:T + 1]
        a_s_li = asd_liT[hd:hd + 1, :]
        alpha_t = jnp.concatenate(
            [s_m1, s_0, s_p1, jnp.broadcast_to(a_s_li, (T, N - T))], axis=1)
        alpha_t = a_d_col[0:T] + alpha_t
        alpha_t = jnp.where(alpha_t >= 0, alpha_t, NEG_SLOPE * alpha_t)
        amax_t = jnp.max(jnp.where(mask_text > 0, alpha_t, -1e30),
                         axis=1, keepdims=True)
        ex_t = mask_text * jnp.exp(jnp.minimum(alpha_t - amax_t, 0.0))
        den_t = jnp.sum(ex_t, axis=1, keepdims=True)
        P_t = ex_t / (den_t + 1e-16)
        band_parts.append(P_t[:, 0:1] * h_dn[:, c0:c0 + out_ch]
                          + P_t[:, 1:2] * h[0:T, c0:c0 + out_ch]
                          + P_t[:, 2:3] * h_up[:, c0:c0 + out_ch])
        pli_parts.append(P_t[:, 3:3 + (N - T)])
        alpha_r = a_s_col + asd_liT[HEADS + hd:HEADS + hd + 1, :]
        alpha_r = jnp.where(alpha_r >= 0, alpha_r, NEG_SLOPE * alpha_r)
        amax_r = jnp.max(jnp.where(colmask_li > 0, alpha_r, -1e30),
                         axis=0, keepdims=True)
        ex_r = colmask_li * jnp.exp(jnp.minimum(alpha_r - amax_r, 0.0))
        den_r = jnp.sum(ex_r, axis=0, keepdims=True)
        P_r = ex_r / (den_r + 1e-16)
        aggr_parts.append(
            jax.lax.dot_general(P_r, h[:, c0:c0 + out_ch],
                                (((0,), (0,)), ((), ())),
                                preferred_element_type=jnp.float32))
    P_li_all = jnp.concatenate(pli_parts, axis=1)
    agg_text = (jnp.concatenate(band_parts, axis=1)
                + jnp.dot(P_li_all, li_scr[...],
                          preferred_element_type=jnp.float32))
    agg_r = jnp.concatenate(aggr_parts, axis=1)
    agg = jnp.concatenate([agg_text, agg_r], axis=0)
    out = jnp.maximum(agg + b, 0.0)
    y = out + x
    mu = jnp.mean(y, axis=1, keepdims=True)
    yc = y - mu
    var = jnp.mean(yc * yc, axis=1, keepdims=True)
    return yc / jnp.sqrt(var + 1e-5) * g + be


def _fused_kernel(t_ref, l_ref, i_ref,
                  w0_ref, b0_ref, g0_ref, be0_ref,
                  w1_ref, b1_ref, g1_ref, be1_ref,
                  w2_ref, b2_ref, g2_ref, be2_ref,
                  o_ref, li_scr, *, T, L, I, H):
    N = T + L + I
    t = t_ref[0]
    lab = l_ref[0]
    img = i_ref[0]
    labn = _rownorm(lab)
    imgn = _rownorm(img)
    dn = (((1,), (1,)), ((), ()))
    sim_l = jax.lax.dot_general(t, labn, dn, preferred_element_type=jnp.float32)
    sim_i = jax.lax.dot_general(t, imgn, dn, preferred_element_type=jnp.float32)
    mask_l = _topk_mask(sim_l, L)
    mask_i = _topk_mask(sim_i, I)
    eye_l = (jax.lax.broadcasted_iota(jnp.int32, (L, L), 0)
             == jax.lax.broadcasted_iota(jnp.int32, (L, L), 1)).astype(jnp.float32)
    eye_i = (jax.lax.broadcasted_iota(jnp.int32, (I, I), 0)
             == jax.lax.broadcasted_iota(jnp.int32, (I, I), 1)).astype(jnp.float32)
    tcol = jax.lax.broadcasted_iota(jnp.int32, (T, 1), 0)
    m_m1 = (tcol >= 1).astype(jnp.float32)
    m_0 = jnp.ones((T, 1), jnp.float32)
    m_p1 = (tcol <= T - 2).astype(jnp.float32)
    mask_text = jnp.concatenate([m_m1, m_0, m_p1, mask_l, mask_i], axis=1)
    ones_ll = jnp.ones((L, L), jnp.float32)
    ones_ii = jnp.ones((I, I), jnp.float32)
    bot = jnp.concatenate([
        jnp.concatenate([ones_ll + eye_l, jnp.ones((L, I), jnp.float32)],
                        axis=1),
        jnp.concatenate([jnp.ones((I, L), jnp.float32), ones_ii + eye_i],
                        axis=1)], axis=0)
    colmask_li = jnp.concatenate(
        [jnp.concatenate([mask_l, mask_i], axis=1), bot], axis=0)

    x = jnp.concatenate([t, lab, img], axis=0)
    plist = [
        (w0_ref, b0_ref, g0_ref, be0_ref),
        (w1_ref, b1_ref, g1_ref, be1_ref),
        (w2_ref, b2_ref, g2_ref, be2_ref),
    ]
    for (w, b, g, be) in plist:
        x = _gat_layer_band2(x, mask_text, colmask_li, w[...], b[...],
                             g[...], be[...], li_scr, T, N, H)
    o_ref[0] = x[0:T, :]


def _run_fused(text_repr, label_repr, image_repr, params, interpret=False):
    B, T, H = text_repr.shape
    L = label_repr.shape[1]
    I = image_repr.shape[1]
    N = T + L + I
    out_ch = H // HEADS
    eyeH = jnp.eye(HEADS, dtype=jnp.float32)
    wspec = pl.BlockSpec((H, H + 2 * HEADS), lambda b: (0, 0))
    vspec = pl.BlockSpec((1, H), lambda b: (0, 0))
    in_specs = [
        pl.BlockSpec((1, T, H), lambda b: (b, 0, 0)),
        pl.BlockSpec((1, L, H), lambda b: (b, 0, 0)),
        pl.BlockSpec((1, I, H), lambda b: (b, 0, 0)),
    ]
    args = [text_repr, label_repr, image_repr]
    for (W, a_s, a_d, b, g, be) in params:
        As = (eyeH[:, None, :] * a_s[:, :, None]).reshape(H, HEADS)
        Ad = (eyeH[:, None, :] * a_d[:, :, None]).reshape(H, HEADS)
        Msd = jnp.concatenate([As, Ad], axis=1)
        Wext = jnp.concatenate([W, W @ Msd], axis=1)
        in_specs += [wspec, vspec, vspec, vspec]
        args += [Wext, b.reshape(1, -1), g.reshape(1, -1), be.reshape(1, -1)]
    return pl.pallas_call(
        functools.partial(_fused_kernel, T=T, L=L, I=I, H=H),
        grid=(B,),
        in_specs=in_specs,
        out_specs=pl.BlockSpec((1, T, H), lambda b: (b, 0, 0)),
        out_shape=jax.ShapeDtypeStruct((B, T, H), jnp.float32),
        scratch_shapes=[pltpu.VMEM((HEADS * (N - T), H), jnp.float32)],
        interpret=interpret,
    )(*args)


def _run(text_repr, label_repr, image_repr, params, interpret=False):
    B, T, H = text_repr.shape
    x = jnp.concatenate([text_repr, label_repr, image_repr], axis=1)
    A = _build_A(text_repr, label_repr, image_repr, interpret=interpret)
    for (W, a_s, a_d, b, g, be) in params:
        x = _layer(x, A, W, a_s, a_d, b.reshape(1, -1), g.reshape(1, -1),
                   be.reshape(1, -1), interpret=interpret)
    return x[:, :T, :]


def kernel(text_repr, label_repr, image_repr,
           W0, as0, ad0, b0, g0, be0,
           W1, as1, ad1, b1, g1, be1,
           W2, as2, ad2, b2, g2, be2):
    params = [
        (W0, as0, ad0, b0, g0, be0),
        (W1, as1, ad1, b1, g1, be1),
        (W2, as2, ad2, b2, g2, be2),
    ]
    return _run_fused(text_repr, label_repr, image_repr, params)


# speedup vs baseline: 1.0797x; 1.0797x over previous
import functools

import jax
import jax.numpy as jnp
from jax.experimental import pallas as pl
from jax.experimental.pallas import tpu as pltpu

HEADS = 4
TOPK = 3
NEG_SLOPE = 0.2


def _rownorm(v):
    n = jnp.sqrt(jnp.sum(v * v, axis=-1, keepdims=True))
    return v / jnp.maximum(n, 1e-8)


def _topk_mask(sim, L):
    cols = [sim[:, j:j + 1] for j in range(L)]
    outs = []
    for j in range(L):
        r = jnp.zeros_like(cols[0])
        for k in range(L):
            if k == j:
                continue
            if k < j:
                beat = cols[k] >= cols[j]
            else:
                beat = cols[k] > cols[j]
            r = r + beat.astype(jnp.float32)
        outs.append((r < (TOPK - 0.5)).astype(jnp.float32))
    return jnp.concatenate(outs, axis=1)


def _edges_kernel(t_ref, l_ref, i_ref, a_ref, *, T, L, I):
    t = t_ref[0]
    lab = l_ref[0]
    img = i_ref[0]
    tn = _rownorm(t)
    labn = _rownorm(lab)
    imgn = _rownorm(img)
    dn = (((1,), (1,)), ((), ()))
    sim_l = jax.lax.dot_general(tn, labn, dn, preferred_element_type=jnp.float32)
    sim_i = jax.lax.dot_general(tn, imgn, dn, preferred_element_type=jnp.float32)
    mask_l = _topk_mask(sim_l, L)
    mask_i = _topk_mask(sim_i, I)
    eye_l = (jax.lax.broadcasted_iota(jnp.int32, (L, L), 0)
             == jax.lax.broadcasted_iota(jnp.int32, (L, L), 1)).astype(jnp.float32)
    mask_lT = jax.lax.dot_general(eye_l, mask_l, dn,
                                  preferred_element_type=jnp.float32)
    eye_i = (jax.lax.broadcasted_iota(jnp.int32, (I, I), 0)
             == jax.lax.broadcasted_iota(jnp.int32, (I, I), 1)).astype(jnp.float32)
    mask_iT = jax.lax.dot_general(eye_i, mask_i, dn,
                                  preferred_element_type=jnp.float32)

    r = jax.lax.broadcasted_iota(jnp.int32, (T, T), 0)
    c = jax.lax.broadcasted_iota(jnp.int32, (T, T), 1)
    chain = ((r - c == 1) | (c - r == 1) | (r == c)).astype(jnp.float32)

    ones_ll = jnp.ones((L, L), jnp.float32)
    ones_ii = jnp.ones((I, I), jnp.float32)
    ones_il = jnp.ones((I, L), jnp.float32)
    ones_li = jnp.ones((L, I), jnp.float32)

    a_ref[0, 0:T, 0:T] = chain
    a_ref[0, 0:T, T:T + L] = mask_l
    a_ref[0, 0:T, T + L:T + L + I] = mask_i
    a_ref[0, T:T + L, 0:T] = mask_lT
    a_ref[0, T:T + L, T:T + L] = ones_ll + eye_l
    a_ref[0, T:T + L, T + L:T + L + I] = ones_li
    a_ref[0, T + L:T + L + I, 0:T] = mask_iT
    a_ref[0, T + L:T + L + I, T:T + L] = ones_il
    a_ref[0, T + L:T + L + I, T + L:T + L + I] = ones_ii + eye_i


def _layer_kernel(x_ref, a_ref, w_ref, as_ref, ad_ref, b_ref, g_ref, be_ref,
                  o_ref, *, N, H):
    out_ch = H // HEADS
    x = x_ref[0]
    A = a_ref[0]
    h = jnp.dot(x, w_ref[...], preferred_element_type=jnp.float32)
    dn = (((1,), (1,)), ((), ()))
    aggs = []
    for hd in range(HEADS):
        hh = h[:, hd * out_ch:(hd + 1) * out_ch]
        asr = as_ref[hd:hd + 1, :]
        adr = ad_ref[hd:hd + 1, :]
        a_src = jax.lax.dot_general(asr, hh, dn,
                                    preferred_element_type=jnp.float32)
        a_dst = jax.lax.dot_general(hh, adr, dn,
                                    preferred_element_type=jnp.float32)
        alpha = a_dst + a_src
        alpha = jnp.where(alpha >= 0, alpha, NEG_SLOPE * alpha)
        malpha = jnp.where(A > 0, alpha, -1e30)
        amax = jnp.max(malpha, axis=1, keepdims=True)
        ex = A * jnp.exp(jnp.minimum(alpha - amax, 0.0))
        den = jnp.sum(ex, axis=1, keepdims=True)
        P = ex / (den + 1e-16)
        aggs.append(jnp.dot(P, hh, preferred_element_type=jnp.float32))
    agg = jnp.concatenate(aggs, axis=1)
    out = jnp.maximum(agg + b_ref[...], 0.0)
    y = out + x
    mu = jnp.mean(y, axis=1, keepdims=True)
    yc = y - mu
    var = jnp.mean(yc * yc, axis=1, keepdims=True)
    o_ref[0] = yc / jnp.sqrt(var + 1e-5) * g_ref[...] + be_ref[...]


def _build_A(text, label, image, interpret=False):
    B, T, H = text.shape
    L = label.shape[1]
    I = image.shape[1]
    N = T + L + I
    return pl.pallas_call(
        functools.partial(_edges_kernel, T=T, L=L, I=I),
        grid=(B,),
        in_specs=[
            pl.BlockSpec((1, T, H), lambda b: (b, 0, 0)),
            pl.BlockSpec((1, L, H), lambda b: (b, 0, 0)),
            pl.BlockSpec((1, I, H), lambda b: (b, 0, 0)),
        ],
        out_specs=pl.BlockSpec((1, N, N), lambda b: (b, 0, 0)),
        out_shape=jax.ShapeDtypeStruct((B, N, N), jnp.float32),
        interpret=interpret,
    )(text, label, image)


def _layer(x, A, W, a_s, a_d, b, g, be, interpret=False):
    B, N, H = x.shape
    out_ch = H // HEADS
    return pl.pallas_call(
        functools.partial(_layer_kernel, N=N, H=H),
        grid=(B,),
        in_specs=[
            pl.BlockSpec((1, N, H), lambda b: (b, 0, 0)),
            pl.BlockSpec((1, N, N), lambda b: (b, 0, 0)),
            pl.BlockSpec((H, H), lambda b: (0, 0)),
            pl.BlockSpec((HEADS, out_ch), lambda b: (0, 0)),
            pl.BlockSpec((HEADS, out_ch), lambda b: (0, 0)),
            pl.BlockSpec((1, H), lambda b: (0, 0)),
            pl.BlockSpec((1, H), lambda b: (0, 0)),
            pl.BlockSpec((1, H), lambda b: (0, 0)),
        ],
        out_specs=pl.BlockSpec((1, N, H), lambda b: (b, 0, 0)),
        out_shape=jax.ShapeDtypeStruct((B, N, H), jnp.float32),
        interpret=interpret,
    )(x, A, W, a_s, a_d, b, g, be)


def _edge_mask(t, lab, img, T, L, I):
    tn = _rownorm(t)
    labn = _rownorm(lab)
    imgn = _rownorm(img)
    dn = (((1,), (1,)), ((), ()))
    sim_l = jax.lax.dot_general(tn, labn, dn, preferred_element_type=jnp.float32)
    sim_i = jax.lax.dot_general(tn, imgn, dn, preferred_element_type=jnp.float32)
    mask_l = _topk_mask(sim_l, L)
    mask_i = _topk_mask(sim_i, I)
    eye_l = (jax.lax.broadcasted_iota(jnp.int32, (L, L), 0)
             == jax.lax.broadcasted_iota(jnp.int32, (L, L), 1)).astype(jnp.float32)
    mask_lT = jax.lax.dot_general(eye_l, mask_l, dn,
                                  preferred_element_type=jnp.float32)
    eye_i = (jax.lax.broadcasted_iota(jnp.int32, (I, I), 0)
             == jax.lax.broadcasted_iota(jnp.int32, (I, I), 1)).astype(jnp.float32)
    mask_iT = jax.lax.dot_general(eye_i, mask_i, dn,
                                  preferred_element_type=jnp.float32)
    r = jax.lax.broadcasted_iota(jnp.int32, (T, T), 0)
    c = jax.lax.broadcasted_iota(jnp.int32, (T, T), 1)
    chain = ((r - c == 1) | (c - r == 1) | (r == c)).astype(jnp.float32)
    return chain, mask_l, mask_i, mask_lT, mask_iT, eye_l, eye_i


def _gat_layer(x, A, W, a_src_w, a_dst_w, b, g, be, N, H):
    out_ch = H // HEADS
    h = jnp.dot(x, W, preferred_element_type=jnp.float32)
    dn = (((1,), (1,)), ((), ()))
    aggs = []
    for hd in range(HEADS):
        hh = h[:, hd * out_ch:(hd + 1) * out_ch]
        asr = a_src_w[hd:hd + 1, :]
        adr = a_dst_w[hd:hd + 1, :]
        a_s = jax.lax.dot_general(asr, hh, dn,
                                  preferred_element_type=jnp.float32)
        a_d = jax.lax.dot_general(hh, adr, dn,
                                  preferred_element_type=jnp.float32)
        alpha = a_d + a_s
        alpha = jnp.where(alpha >= 0, alpha, NEG_SLOPE * alpha)
        malpha = jnp.where(A > 0, alpha, -1e30)
        amax = jnp.max(malpha, axis=1, keepdims=True)
        ex = A * jnp.exp(jnp.minimum(alpha - amax, 0.0))
        den = jnp.sum(ex, axis=1, keepdims=True)
        P = ex / (den + 1e-16)
        aggs.append(jnp.dot(P, hh, preferred_element_type=jnp.float32))
    agg = jnp.concatenate(aggs, axis=1)
    out = jnp.maximum(agg + b, 0.0)
    y = out + x
    mu = jnp.mean(y, axis=1, keepdims=True)
    yc = y - mu
    var = jnp.mean(yc * yc, axis=1, keepdims=True)
    return yc / jnp.sqrt(var + 1e-5) * g + be


def _gat_layer_band(x, mask_text, rowmask_li, W, a_src_w, a_dst_w, b, g, be,
                    T, N, H):
    out_ch = H // HEADS
    h = jnp.dot(x, W, preferred_element_type=jnp.float32)
    dn = (((1,), (1,)), ((), ()))
    aggs = []
    for hd in range(HEADS):
        hh = h[:, hd * out_ch:(hd + 1) * out_ch]
        asr = a_src_w[hd:hd + 1, :]
        adr = a_dst_w[hd:hd + 1, :]
        a_s_col = jax.lax.dot_general(hh, asr, dn,
                                      preferred_element_type=jnp.float32)
        a_d_col = jax.lax.dot_general(hh, adr, dn,
                                      preferred_element_type=jnp.float32)
        s_0 = a_s_col[0:T]
        s_m1 = jnp.concatenate([a_s_col[0:1], a_s_col[0:T - 1]], axis=0)
        s_p1 = a_s_col[1:T + 1]
        a_s_li = jax.lax.dot_general(asr, hh[T:N], dn,
                                     preferred_element_type=jnp.float32)
        alpha_t = jnp.concatenate(
            [s_m1, s_0, s_p1, jnp.broadcast_to(a_s_li, (T, N - T))], axis=1)
        alpha_t = a_d_col[0:T] + alpha_t
        alpha_t = jnp.where(alpha_t >= 0, alpha_t, NEG_SLOPE * alpha_t)
        amax_t = jnp.max(jnp.where(mask_text > 0, alpha_t, -1e30),
                         axis=1, keepdims=True)
        ex_t = mask_text * jnp.exp(jnp.minimum(alpha_t - amax_t, 0.0))
        den_t = jnp.sum(ex_t, axis=1, keepdims=True)
        P_t = ex_t / (den_t + 1e-16)
        h_m1 = jnp.concatenate([hh[0:1], hh[0:T - 1]], axis=0)
        h_p1 = hh[1:T + 1]
        agg_t = (P_t[:, 0:1] * h_m1 + P_t[:, 1:2] * hh[0:T]
                 + P_t[:, 2:3] * h_p1
                 + jnp.dot(P_t[:, 3:3 + (N - T)], hh[T:N],
                           preferred_element_type=jnp.float32))
        a_s_row = jax.lax.dot_general(asr, hh, dn,
                                      preferred_element_type=jnp.float32)
        alpha_r = a_d_col[T:N] + a_s_row
        alpha_r = jnp.where(alpha_r >= 0, alpha_r, NEG_SLOPE * alpha_r)
        amax_r = jnp.max(jnp.where(rowmask_li > 0, alpha_r, -1e30),
                         axis=1, keepdims=True)
        ex_r = rowmask_li * jnp.exp(jnp.minimum(alpha_r - amax_r, 0.0))
        den_r = jnp.sum(ex_r, axis=1, keepdims=True)
        P_r = ex_r / (den_r + 1e-16)
        agg_r = jnp.dot(P_r, hh, preferred_element_type=jnp.float32)
        aggs.append(jnp.concatenate([agg_t, agg_r], axis=0))
    agg = jnp.concatenate(aggs, axis=1)
    out = jnp.maximum(agg + b, 0.0)
    y = out + x
    mu = jnp.mean(y, axis=1, keepdims=True)
    yc = y - mu
    var = jnp.mean(yc * yc, axis=1, keepdims=True)
    return yc / jnp.sqrt(var + 1e-5) * g + be


def _gat_layer_band2(x, m_m1, m_p1, mask_li_text, colmask_li, W, Msd,
                     b, g, be, li_scr, T, N, H):
    out_ch = H // HEADS
    LI = N - T
    h = jnp.dot(x, W, preferred_element_type=jnp.float32)
    asd = jnp.dot(h, Msd, preferred_element_type=jnp.float32)
    eye8 = (jax.lax.broadcasted_iota(jnp.int32, (2 * HEADS, 2 * HEADS), 0)
            == jax.lax.broadcasted_iota(jnp.int32,
                                        (2 * HEADS, 2 * HEADS), 1)
            ).astype(jnp.float32)
    asd_liT = jax.lax.dot_general(asd[T:N], eye8, (((0,), (0,)), ((), ())),
                                  preferred_element_type=jnp.float32)
    h_dn = jnp.concatenate([h[0:1], h[0:T - 1]], axis=0)
    h_up = h[1:T + 1]
    li_scr[...] = jnp.zeros((HEADS * LI, H), jnp.float32)
    for hd in range(HEADS):
        c0 = hd * out_ch
        li_scr[hd * LI:(hd + 1) * LI, c0:c0 + out_ch] = h[T:N, c0:c0 + out_ch]
    band_parts = []
    pli_parts = []
    aggr_parts = []
    NEGB = jnp.float32(-1e30)
    for hd in range(HEADS):
        c0 = hd * out_ch
        a_s_col = asd[:, hd:hd + 1]
        a_d_col = asd[:, HEADS + hd:HEADS + hd + 1]
        a_d_t = a_d_col[0:T]
        s_m1 = jnp.concatenate([a_s_col[0:1], a_s_col[0:T - 1]], axis=0)
        al0 = a_d_t + s_m1
        al1 = a_d_t + a_s_col[0:T]
        al2 = a_d_t + a_s_col[1:T + 1]
        alli = a_d_t + asd_liT[hd:hd + 1, :]
        al0 = jnp.where(al0 >= 0, al0, NEG_SLOPE * al0)
        al1 = jnp.where(al1 >= 0, al1, NEG_SLOPE * al1)
        al2 = jnp.where(al2 >= 0, al2, NEG_SLOPE * al2)
        alli = jnp.where(alli >= 0, alli, NEG_SLOPE * alli)
        amax = jnp.maximum(
            jnp.maximum(jnp.where(m_m1 > 0, al0, NEGB), al1),
            jnp.maximum(jnp.where(m_p1 > 0, al2, NEGB),
                        jnp.max(jnp.where(mask_li_text > 0, alli, NEGB),
                                axis=1, keepdims=True)))
        ex0 = m_m1 * jnp.exp(jnp.minimum(al0 - amax, 0.0))
        ex1 = jnp.exp(jnp.minimum(al1 - amax, 0.0))
        ex2 = m_p1 * jnp.exp(jnp.minimum(al2 - amax, 0.0))
        exli = mask_li_text * jnp.exp(jnp.minimum(alli - amax, 0.0))
        rden = 1.0 / (ex0 + ex1 + ex2
                      + jnp.sum(exli, axis=1, keepdims=True) + 1e-16)
        band_parts.append(
            ((ex0 * rden) * h_dn[:, c0:c0 + out_ch]
             + (ex1 * rden) * h[0:T, c0:c0 + out_ch]
             + (ex2 * rden) * h_up[:, c0:c0 + out_ch]))
        pli_parts.append(exli * rden)
        alpha_r = a_s_col + asd_liT[HEADS + hd:HEADS + hd + 1, :]
        alpha_r = jnp.where(alpha_r >= 0, alpha_r, NEG_SLOPE * alpha_r)
        amax_r = jnp.max(jnp.where(colmask_li > 0, alpha_r, NEGB),
                         axis=0, keepdims=True)
        ex_r = colmask_li * jnp.exp(jnp.minimum(alpha_r - amax_r, 0.0))
        den_r = jnp.sum(ex_r, axis=0, keepdims=True)
        P_r = ex_r / (den_r + 1e-16)
        aggr_parts.append(
            jax.lax.dot_general(P_r, h[:, c0:c0 + out_ch],
                                (((0,), (0,)), ((), ())),
                                preferred_element_type=jnp.float32))
    P_li_all = jnp.concatenate(pli_parts, axis=1)
    agg_text = (jnp.concatenate(band_parts, axis=1)
                + jnp.dot(P_li_all, li_scr[...],
                          preferred_element_type=jnp.float32))
    agg_r = jnp.concatenate(aggr_parts, axis=1)
    agg = jnp.concatenate([agg_text, agg_r], axis=0)
    out = jnp.maximum(agg + b, 0.0)
    y = out + x
    mu = jnp.mean(y, axis=1, keepdims=True)
    yc = y - mu
    var = jnp.mean(yc * yc, axis=1, keepdims=True)
    return yc / jnp.sqrt(var + 1e-5) * g + be


def _fused_kernel(t_ref, l_ref, i_ref,
                  w0_ref, m0_ref, b0_ref, g0_ref, be0_ref,
                  w1_ref, m1_ref, b1_ref, g1_ref, be1_ref,
                  w2_ref, m2_ref, b2_ref, g2_ref, be2_ref,
                  o_ref, li_scr, *, T, L, I, H):
    N = T + L + I
    t = t_ref[0]
    lab = l_ref[0]
    img = i_ref[0]
    tn = _rownorm(t)
    labn = _rownorm(lab)
    imgn = _rownorm(img)
    dn = (((1,), (1,)), ((), ()))
    sim_l = jax.lax.dot_general(tn, labn, dn, preferred_element_type=jnp.float32)
    sim_i = jax.lax.dot_general(tn, imgn, dn, preferred_element_type=jnp.float32)
    mask_l = _topk_mask(sim_l, L)
    mask_i = _topk_mask(sim_i, I)
    eye_l = (jax.lax.broadcasted_iota(jnp.int32, (L, L), 0)
             == jax.lax.broadcasted_iota(jnp.int32, (L, L), 1)).astype(jnp.float32)
    eye_i = (jax.lax.broadcasted_iota(jnp.int32, (I, I), 0)
             == jax.lax.broadcasted_iota(jnp.int32, (I, I), 1)).astype(jnp.float32)
    tcol = jax.lax.broadcasted_iota(jnp.int32, (T, 1), 0)
    m_m1 = (tcol >= 1).astype(jnp.float32)
    m_p1 = (tcol <= T - 2).astype(jnp.float32)
    mask_li_text = jnp.concatenate([mask_l, mask_i], axis=1)
    ones_ll = jnp.ones((L, L), jnp.float32)
    ones_ii = jnp.ones((I, I), jnp.float32)
    bot = jnp.concatenate([
        jnp.concatenate([ones_ll + eye_l, jnp.ones((L, I), jnp.float32)],
                        axis=1),
        jnp.concatenate([jnp.ones((I, L), jnp.float32), ones_ii + eye_i],
                        axis=1)], axis=0)
    colmask_li = jnp.concatenate([mask_li_text, bot], axis=0)

    x = jnp.concatenate([t, lab, img], axis=0)
    plist = [
        (w0_ref, m0_ref, b0_ref, g0_ref, be0_ref),
        (w1_ref, m1_ref, b1_ref, g1_ref, be1_ref),
        (w2_ref, m2_ref, b2_ref, g2_ref, be2_ref),
    ]
    for (w, m, b, g, be) in plist:
        x = _gat_layer_band2(x, m_m1, m_p1, mask_li_text, colmask_li,
                             w[...], m[...], b[...], g[...], be[...],
                             li_scr, T, N, H)
    o_ref[0] = x[0:T, :]


def _run_fused(text_repr, label_repr, image_repr, params, interpret=False):
    B, T, H = text_repr.shape
    L = label_repr.shape[1]
    I = image_repr.shape[1]
    N = T + L + I
    out_ch = H // HEADS
    eyeH = jnp.eye(HEADS, dtype=jnp.float32)
    wspec = pl.BlockSpec((H, H), lambda b: (0, 0))
    mspec = pl.BlockSpec((H, 2 * HEADS), lambda b: (0, 0))
    vspec = pl.BlockSpec((1, H), lambda b: (0, 0))
    in_specs = [
        pl.BlockSpec((1, T, H), lambda b: (b, 0, 0)),
        pl.BlockSpec((1, L, H), lambda b: (b, 0, 0)),
        pl.BlockSpec((1, I, H), lambda b: (b, 0, 0)),
    ]
    args = [text_repr, label_repr, image_repr]
    for (W, a_s, a_d, b, g, be) in params:
        As = (eyeH[:, None, :] * a_s[:, :, None]).reshape(H, HEADS)
        Ad = (eyeH[:, None, :] * a_d[:, :, None]).reshape(H, HEADS)
        Msd = jnp.concatenate([As, Ad], axis=1)
        in_specs += [wspec, mspec, vspec, vspec, vspec]
        args += [W, Msd, b.reshape(1, -1), g.reshape(1, -1), be.reshape(1, -1)]
    return pl.pallas_call(
        functools.partial(_fused_kernel, T=T, L=L, I=I, H=H),
        grid=(B,),
        in_specs=in_specs,
        out_specs=pl.BlockSpec((1, T, H), lambda b: (b, 0, 0)),
        out_shape=jax.ShapeDtypeStruct((B, T, H), jnp.float32),
        scratch_shapes=[pltpu.VMEM((HEADS * (N - T), H), jnp.float32)],
        interpret=interpret,
    )(*args)


def _run(text_repr, label_repr, image_repr, params, interpret=False):
    B, T, H = text_repr.shape
    x = jnp.concatenate([text_repr, label_repr, image_repr], axis=1)
    A = _build_A(text_repr, label_repr, image_repr, interpret=interpret)
    for (W, a_s, a_d, b, g, be) in params:
        x = _layer(x, A, W, a_s, a_d, b.reshape(1, -1), g.reshape(1, -1),
                   be.reshape(1, -1), interpret=interpret)
    return x[:, :T, :]


def kernel(text_repr, label_repr, image_repr,
           W0, as0, ad0, b0, g0, be0,
           W1, as1, ad1, b1, g1, be1,
           W2, as2, ad2, b2, g2, be2):
    params = [
        (W0, as0, ad0, b0, g0, be0),
        (W1, as1, ad1, b1, g1, be1),
        (W2, as2, ad2, b2, g2, be2),
    ]
    return _run_fused(text_repr, label_repr, image_repr, params)


# speedup vs baseline: 1.3065x; 1.2101x over previous
import functools

import jax
import jax.numpy as jnp
from jax.experimental import pallas as pl
from jax.experimental.pallas import tpu as pltpu

HEADS = 4
TOPK = 3
NEG_SLOPE = 0.2


def _rownorm(v):
    n = jnp.sqrt(jnp.sum(v * v, axis=-1, keepdims=True))
    return v / jnp.maximum(n, 1e-8)


def _topk_mask(sim, L):
    cols = [sim[:, j:j + 1] for j in range(L)]
    outs = []
    for j in range(L):
        r = jnp.zeros_like(cols[0])
        for k in range(L):
            if k == j:
                continue
            if k < j:
                beat = cols[k] >= cols[j]
            else:
                beat = cols[k] > cols[j]
            r = r + beat.astype(jnp.float32)
        outs.append((r < (TOPK - 0.5)).astype(jnp.float32))
    return jnp.concatenate(outs, axis=1)


def _fused_kernel(t_ref, l_ref, i_ref,
                  w0_ref, m0_ref, b0_ref, g0_ref, be0_ref,
                  w1_ref, m1_ref, b1_ref, g1_ref, be1_ref,
                  w2_ref, m2_ref, b2_ref, g2_ref, be2_ref,
                  o_ref, a_scr, an_scr, *, T, L, I, H):
    N = T + L + I
    out_ch = H // HEADS
    t = t_ref[0]
    lab = l_ref[0]
    img = i_ref[0]
    tn = _rownorm(t)
    labn = _rownorm(lab)
    imgn = _rownorm(img)
    dn = (((1,), (1,)), ((), ()))
    sim_l = jax.lax.dot_general(tn, labn, dn, preferred_element_type=jnp.float32)
    sim_i = jax.lax.dot_general(tn, imgn, dn, preferred_element_type=jnp.float32)
    mask_l = _topk_mask(sim_l, L)
    mask_i = _topk_mask(sim_i, I)
    eye_l = (jax.lax.broadcasted_iota(jnp.int32, (L, L), 0)
             == jax.lax.broadcasted_iota(jnp.int32, (L, L), 1)).astype(jnp.float32)
    mask_lT = jax.lax.dot_general(eye_l, mask_l, dn,
                                  preferred_element_type=jnp.float32)
    eye_i = (jax.lax.broadcasted_iota(jnp.int32, (I, I), 0)
             == jax.lax.broadcasted_iota(jnp.int32, (I, I), 1)).astype(jnp.float32)
    mask_iT = jax.lax.dot_general(eye_i, mask_i, dn,
                                  preferred_element_type=jnp.float32)
    r = jax.lax.broadcasted_iota(jnp.int32, (T, T), 0)
    c = jax.lax.broadcasted_iota(jnp.int32, (T, T), 1)
    chain = ((r - c == 1) | (c - r == 1) | (r == c)).astype(jnp.float32)
    ones_ll = jnp.ones((L, L), jnp.float32)
    ones_ii = jnp.ones((I, I), jnp.float32)
    a_scr[0:T, 0:T] = chain
    a_scr[0:T, T:T + L] = mask_l
    a_scr[0:T, T + L:N] = mask_i
    a_scr[T:T + L, 0:T] = mask_lT
    a_scr[T:T + L, T:T + L] = ones_ll + eye_l
    a_scr[T:T + L, T + L:N] = jnp.ones((L, I), jnp.float32)
    a_scr[T + L:N, 0:T] = mask_iT
    a_scr[T + L:N, T:T + L] = jnp.ones((I, L), jnp.float32)
    a_scr[T + L:N, T + L:N] = ones_ii + eye_i
    A = a_scr[...]
    an_scr[...] = jnp.where(A > 0, 0.0, -1e30)
    Aneg = an_scr[...]

    x = jnp.concatenate([t, lab, img], axis=0)
    plist = [
        (w0_ref, m0_ref, b0_ref, g0_ref, be0_ref),
        (w1_ref, m1_ref, b1_ref, g1_ref, be1_ref),
        (w2_ref, m2_ref, b2_ref, g2_ref, be2_ref),
    ]
    for (w_ref, m_ref, b_ref, g_ref, be_ref) in plist:
        h = jnp.dot(x, w_ref[...], preferred_element_type=jnp.float32)
        asd = jnp.dot(h, m_ref[...], preferred_element_type=jnp.float32)
        aggs = []
        for hd in range(HEADS):
            hh = h[:, hd * out_ch:(hd + 1) * out_ch]
            a_s_row = jax.lax.dot_general(
                jnp.ones((1, 1), jnp.float32), asd[:, hd:hd + 1], dn,
                preferred_element_type=jnp.float32)
            a_d_col = asd[:, HEADS + hd:HEADS + hd + 1]
            alpha = a_d_col + a_s_row
            alpha = jnp.maximum(alpha, NEG_SLOPE * alpha)
            malpha = alpha + Aneg
            amax = jnp.max(malpha, axis=1, keepdims=True)
            ex = A * jnp.exp(malpha - amax)
            rden = 1.0 / (jnp.sum(ex, axis=1, keepdims=True) + 1e-16)
            aggs.append(jnp.dot(ex, hh,
                                preferred_element_type=jnp.float32) * rden)
        agg = jnp.concatenate(aggs, axis=1)
        out = jnp.maximum(agg + b_ref[...], 0.0)
        y = out + x
        mu = jnp.mean(y, axis=1, keepdims=True)
        var = jnp.maximum(jnp.mean(y * y, axis=1, keepdims=True) - mu * mu,
                          0.0)
        x = (y - mu) / jnp.sqrt(var + 1e-5) * g_ref[...] + be_ref[...]
    o_ref[0] = x[0:T, :]


def _run_fused(text_repr, label_repr, image_repr, params, interpret=False):
    B, T, H = text_repr.shape
    L = label_repr.shape[1]
    I = image_repr.shape[1]
    N = T + L + I
    eyeH = jnp.eye(HEADS, dtype=jnp.float32)
    wspec = pl.BlockSpec((H, H), lambda b: (0, 0))
    mspec = pl.BlockSpec((H, 2 * HEADS), lambda b: (0, 0))
    vspec = pl.BlockSpec((1, H), lambda b: (0, 0))
    in_specs = [
        pl.BlockSpec((1, T, H), lambda b: (b, 0, 0)),
        pl.BlockSpec((1, L, H), lambda b: (b, 0, 0)),
        pl.BlockSpec((1, I, H), lambda b: (b, 0, 0)),
    ]
    args = [text_repr, label_repr, image_repr]
    for (W, a_s, a_d, b, g, be) in params:
        As = (eyeH[:, None, :] * a_s[:, :, None]).reshape(H, HEADS)
        Ad = (eyeH[:, None, :] * a_d[:, :, None]).reshape(H, HEADS)
        Msd = jnp.concatenate([As, Ad], axis=1)
        in_specs += [wspec, mspec, vspec, vspec, vspec]
        args += [W, Msd, b.reshape(1, -1), g.reshape(1, -1), be.reshape(1, -1)]
    return pl.pallas_call(
        functools.partial(_fused_kernel, T=T, L=L, I=I, H=H),
        grid=(B,),
        in_specs=in_specs,
        out_specs=pl.BlockSpec((1, T, H), lambda b: (b, 0, 0)),
        out_shape=jax.ShapeDtypeStruct((B, T, H), jnp.float32),
        scratch_shapes=[pltpu.VMEM((N, N), jnp.float32),
                        pltpu.VMEM((N, N), jnp.float32)],
        interpret=interpret,
    )(*args)


def kernel(text_repr, label_repr, image_repr,
           W0, as0, ad0, b0, g0, be0,
           W1, as1, ad1, b1, g1, be1,
           W2, as2, ad2, b2, g2, be2):
    params = [
        (W0, as0, ad0, b0, g0, be0),
        (W1, as1, ad1, b1, g1, be1),
        (W2, as2, ad2, b2, g2, be2),
    ]
    return _run_fused(text_repr, label_repr, image_repr, params)


# speedup vs baseline: 1.3441x; 1.0288x over previous
import functools

import jax
import jax.numpy as jnp
from jax.experimental import pallas as pl
from jax.experimental.pallas import tpu as pltpu

HEADS = 4
TOPK = 3
NEG_SLOPE = 0.2


def _cosnorm(x):
    return x / jnp.clip(jnp.linalg.norm(x, axis=-1, keepdims=True), 1e-8)


def _topk_mask(sim, L):
    cols = [sim[:, j:j + 1] for j in range(L)]
    outs = []
    for j in range(L):
        r = jnp.zeros_like(cols[0])
        for k in range(L):
            if k == j:
                continue
            if k < j:
                beat = cols[k] >= cols[j]
            else:
                beat = cols[k] > cols[j]
            r = r + beat.astype(jnp.float32)
        outs.append((r < (TOPK - 0.5)).astype(jnp.float32))
    return jnp.concatenate(outs, axis=1)


def _fused_kernel(t_ref, l_ref, i_ref, sl_ref, si_ref,
                  w0_ref, as0_ref, ad0_ref, b0_ref, g0_ref, be0_ref,
                  w1_ref, as1_ref, ad1_ref, b1_ref, g1_ref, be1_ref,
                  w2_ref, as2_ref, ad2_ref, b2_ref, g2_ref, be2_ref,
                  o_ref, a_scr, an_scr, *, T, L, I, H):
    N = T + L + I
    out_ch = H // HEADS
    t = t_ref[0]
    lab = l_ref[0]
    img = i_ref[0]
    dn = (((1,), (1,)), ((), ()))
    mask_l = _topk_mask(sl_ref[0], L)
    mask_i = _topk_mask(si_ref[0], I)
    eye_l = (jax.lax.broadcasted_iota(jnp.int32, (L, L), 0)
             == jax.lax.broadcasted_iota(jnp.int32, (L, L), 1)).astype(jnp.float32)
    mask_lT = jax.lax.dot_general(eye_l, mask_l, dn,
                                  preferred_element_type=jnp.float32)
    eye_i = (jax.lax.broadcasted_iota(jnp.int32, (I, I), 0)
             == jax.lax.broadcasted_iota(jnp.int32, (I, I), 1)).astype(jnp.float32)
    mask_iT = jax.lax.dot_general(eye_i, mask_i, dn,
                                  preferred_element_type=jnp.float32)
    r = jax.lax.broadcasted_iota(jnp.int32, (T, T), 0)
    c = jax.lax.broadcasted_iota(jnp.int32, (T, T), 1)
    chain = ((r - c == 1) | (c - r == 1) | (r == c)).astype(jnp.float32)
    ones_ll = jnp.ones((L, L), jnp.float32)
    ones_ii = jnp.ones((I, I), jnp.float32)
    a_scr[0:T, 0:T] = chain
    a_scr[0:T, T:T + L] = mask_l
    a_scr[0:T, T + L:N] = mask_i
    a_scr[T:T + L, 0:T] = mask_lT
    a_scr[T:T + L, T:T + L] = ones_ll + eye_l
    a_scr[T:T + L, T + L:N] = jnp.ones((L, I), jnp.float32)
    a_scr[T + L:N, 0:T] = mask_iT
    a_scr[T + L:N, T:T + L] = jnp.ones((I, L), jnp.float32)
    a_scr[T + L:N, T + L:N] = ones_ii + eye_i
    A = a_scr[...]
    an_scr[...] = jnp.where(A > 0, 0.0, -1e30)
    Aneg = an_scr[...]

    x = jnp.concatenate([t, lab, img], axis=0)
    plist = [
        (w0_ref, as0_ref, ad0_ref, b0_ref, g0_ref, be0_ref),
        (w1_ref, as1_ref, ad1_ref, b1_ref, g1_ref, be1_ref),
        (w2_ref, as2_ref, ad2_ref, b2_ref, g2_ref, be2_ref),
    ]
    for (w_ref, as_ref, ad_ref, b_ref, g_ref, be_ref) in plist:
        h = jnp.dot(x, w_ref[...], preferred_element_type=jnp.float32)
        aggs = []
        for hd in range(HEADS):
            hh = h[:, hd * out_ch:(hd + 1) * out_ch]
            asr = as_ref[hd:hd + 1, :]
            adr = ad_ref[hd:hd + 1, :]
            a_s_row = jax.lax.dot_general(
                asr, hh, dn, preferred_element_type=jnp.float32)
            a_d_col = jax.lax.dot_general(
                hh, adr, dn, preferred_element_type=jnp.float32)
            alpha = a_d_col + a_s_row
            alpha = jnp.maximum(alpha, NEG_SLOPE * alpha)
            malpha = alpha + Aneg
            amax = jnp.max(malpha, axis=1, keepdims=True)
            ex = A * jnp.exp(malpha - amax)
            rden = 1.0 / (jnp.sum(ex, axis=1, keepdims=True) + 1e-16)
            aggs.append(jnp.dot(ex, hh,
                                preferred_element_type=jnp.float32) * rden)
        agg = jnp.concatenate(aggs, axis=1)
        out = jnp.maximum(agg + b_ref[...], 0.0)
        y = out + x
        mu = jnp.mean(y, axis=1, keepdims=True)
        var = jnp.maximum(jnp.mean(y * y, axis=1, keepdims=True) - mu * mu,
                          0.0)
        x = (y - mu) / jnp.sqrt(var + 1e-5) * g_ref[...] + be_ref[...]
    o_ref[0] = x[0:T, :]


def _run_fused(text_repr, label_repr, image_repr, params, interpret=False):
    B, T, H = text_repr.shape
    L = label_repr.shape[1]
    I = image_repr.shape[1]
    N = T + L + I
    out_ch = H // HEADS
    sims_l = []
    sims_i = []
    for b in range(B):
        tn = _cosnorm(text_repr[b])
        sims_l.append(tn @ _cosnorm(label_repr[b]).T)
        sims_i.append(tn @ _cosnorm(image_repr[b]).T)
    sim_l = jnp.stack(sims_l, axis=0)
    sim_i = jnp.stack(sims_i, axis=0)
    wspec = pl.BlockSpec((H, H), lambda b: (0, 0))
    aspec = pl.BlockSpec((HEADS, out_ch), lambda b: (0, 0))
    vspec = pl.BlockSpec((1, H), lambda b: (0, 0))
    in_specs = [
        pl.BlockSpec((1, T, H), lambda b: (b, 0, 0)),
        pl.BlockSpec((1, L, H), lambda b: (b, 0, 0)),
        pl.BlockSpec((1, I, H), lambda b: (b, 0, 0)),
        pl.BlockSpec((1, T, L), lambda b: (b, 0, 0)),
        pl.BlockSpec((1, T, I), lambda b: (b, 0, 0)),
    ]
    args = [text_repr, label_repr, image_repr, sim_l, sim_i]
    for (W, a_s, a_d, b, g, be) in params:
        in_specs += [wspec, aspec, aspec, vspec, vspec, vspec]
        args += [W, a_s, a_d, b.reshape(1, -1), g.reshape(1, -1),
                 be.reshape(1, -1)]
    return pl.pallas_call(
        functools.partial(_fused_kernel, T=T, L=L, I=I, H=H),
        grid=(B,),
        in_specs=in_specs,
        out_specs=pl.BlockSpec((1, T, H), lambda b: (b, 0, 0)),
        out_shape=jax.ShapeDtypeStruct((B, T, H), jnp.float32),
        scratch_shapes=[pltpu.VMEM((N, N), jnp.float32),
                        pltpu.VMEM((N, N), jnp.float32)],
        interpret=interpret,
    )(*args)


def kernel(text_repr, label_repr, image_repr,
           W0, as0, ad0, b0, g0, be0,
           W1, as1, ad1, b1, g1, be1,
           W2, as2, ad2, b2, g2, be2):
    params = [
        (W0, as0, ad0, b0, g0, be0),
        (W1, as1, ad1, b1, g1, be1),
        (W2, as2, ad2, b2, g2, be2),
    ]
    return _run_fused(text_repr, label_repr, image_repr, params)


# speedup vs baseline: 1.4020x; 1.0430x over previous
import functools

import jax
import jax.numpy as jnp
from jax.experimental import pallas as pl
from jax.experimental.pallas import tpu as pltpu

HEADS = 4
TOPK = 3
NEG_SLOPE = 0.2


def _cosnorm(x):
    return x / jnp.clip(jnp.linalg.norm(x, axis=-1, keepdims=True), 1e-8)


def _topk_mask(sim, L):
    cols = [sim[:, j:j + 1] for j in range(L)]
    outs = []
    for j in range(L):
        r = jnp.zeros_like(cols[0])
        for k in range(L):
            if k == j:
                continue
            if k < j:
                beat = cols[k] >= cols[j]
            else:
                beat = cols[k] > cols[j]
            r = r + beat.astype(jnp.float32)
        outs.append((r < (TOPK - 0.5)).astype(jnp.float32))
    return jnp.concatenate(outs, axis=1)


def _fused_kernel(t_ref, l_ref, i_ref, sl_ref, si_ref,
                  w0_ref, as0_ref, ad0_ref, b0_ref, g0_ref, be0_ref,
                  w1_ref, as1_ref, ad1_ref, b1_ref, g1_ref, be1_ref,
                  w2_ref, as2_ref, ad2_ref, b2_ref, g2_ref, be2_ref,
                  o_ref, a_scr, an_scr, *, T, L, I, H):
    N = T + L + I
    out_ch = H // HEADS
    t = t_ref[0]
    lab = l_ref[0]
    img = i_ref[0]
    dn = (((1,), (1,)), ((), ()))
    mask_l = _topk_mask(sl_ref[0], L)
    mask_i = _topk_mask(si_ref[0], I)
    eye_l = (jax.lax.broadcasted_iota(jnp.int32, (L, L), 0)
             == jax.lax.broadcasted_iota(jnp.int32, (L, L), 1)).astype(jnp.float32)
    mask_lT = jax.lax.dot_general(eye_l, mask_l, dn,
                                  preferred_element_type=jnp.float32)
    eye_i = (jax.lax.broadcasted_iota(jnp.int32, (I, I), 0)
             == jax.lax.broadcasted_iota(jnp.int32, (I, I), 1)).astype(jnp.float32)
    mask_iT = jax.lax.dot_general(eye_i, mask_i, dn,
                                  preferred_element_type=jnp.float32)
    r = jax.lax.broadcasted_iota(jnp.int32, (T, T), 0)
    c = jax.lax.broadcasted_iota(jnp.int32, (T, T), 1)
    chain = ((r - c == 1) | (c - r == 1) | (r == c)).astype(jnp.float32)
    ones_ll = jnp.ones((L, L), jnp.float32)
    ones_ii = jnp.ones((I, I), jnp.float32)
    a_scr[0:T, 0:T] = chain
    a_scr[0:T, T:T + L] = mask_l
    a_scr[0:T, T + L:N] = mask_i
    a_scr[T:T + L, 0:T] = mask_lT
    a_scr[T:T + L, T:T + L] = ones_ll + eye_l
    a_scr[T:T + L, T + L:N] = jnp.ones((L, I), jnp.float32)
    a_scr[T + L:N, 0:T] = mask_iT
    a_scr[T + L:N, T:T + L] = jnp.ones((I, L), jnp.float32)
    a_scr[T + L:N, T + L:N] = ones_ii + eye_i
    A = a_scr[...]
    an_scr[...] = jnp.where(A > 0, 0.0, -1e30)
    Aneg = an_scr[...]

    x = jnp.concatenate([t, lab, img], axis=0)
    plist = [
        (w0_ref, as0_ref, ad0_ref, b0_ref, g0_ref, be0_ref),
        (w1_ref, as1_ref, ad1_ref, b1_ref, g1_ref, be1_ref),
        (w2_ref, as2_ref, ad2_ref, b2_ref, g2_ref, be2_ref),
    ]
    for (w_ref, as_ref, ad_ref, b_ref, g_ref, be_ref) in plist:
        h = jnp.dot(x, w_ref[...], preferred_element_type=jnp.float32)
        aggs = []
        for hd in range(HEADS):
            hh = h[:, hd * out_ch:(hd + 1) * out_ch]
            asr = as_ref[hd:hd + 1, :]
            adr = ad_ref[hd:hd + 1, :]
            a_s_row = jax.lax.dot_general(
                asr, hh, dn, preferred_element_type=jnp.float32)
            a_d_col = jax.lax.dot_general(
                hh, adr, dn, preferred_element_type=jnp.float32)
            alpha = a_d_col + a_s_row
            alpha = jnp.maximum(alpha, NEG_SLOPE * alpha)
            malpha = alpha + Aneg
            amax = jnp.max(malpha, axis=1, keepdims=True)
            ex = A * jnp.exp(malpha - amax)
            rden = 1.0 / (jnp.sum(ex, axis=1, keepdims=True) + 1e-16)
            aggs.append(jnp.dot(ex, hh,
                                preferred_element_type=jnp.float32) * rden)
        agg = jnp.concatenate(aggs, axis=1)
        out = jnp.maximum(agg + b_ref[...], 0.0)
        y = out + x
        mu = jnp.mean(y, axis=1, keepdims=True)
        var = jnp.maximum(jnp.mean(y * y, axis=1, keepdims=True) - mu * mu,
                          0.0)
        x = (y - mu) / jnp.sqrt(var + 1e-5) * g_ref[...] + be_ref[...]
    o_ref[0] = x[0:T, :]


def _run_fused(text_repr, label_repr, image_repr, params, interpret=False):
    B, T, H = text_repr.shape
    L = label_repr.shape[1]
    I = image_repr.shape[1]
    N = T + L + I
    out_ch = H // HEADS
    tn = _cosnorm(text_repr)
    sim_l = jnp.einsum('bth,blh->btl', tn, _cosnorm(label_repr))
    sim_i = jnp.einsum('bth,bih->bti', tn, _cosnorm(image_repr))
    wspec = pl.BlockSpec((H, H), lambda b: (0, 0))
    aspec = pl.BlockSpec((HEADS, out_ch), lambda b: (0, 0))
    vspec = pl.BlockSpec((1, H), lambda b: (0, 0))
    in_specs = [
        pl.BlockSpec((1, T, H), lambda b: (b, 0, 0)),
        pl.BlockSpec((1, L, H), lambda b: (b, 0, 0)),
        pl.BlockSpec((1, I, H), lambda b: (b, 0, 0)),
        pl.BlockSpec((1, T, L), lambda b: (b, 0, 0)),
        pl.BlockSpec((1, T, I), lambda b: (b, 0, 0)),
    ]
    args = [text_repr, label_repr, image_repr, sim_l, sim_i]
    for (W, a_s, a_d, b, g, be) in params:
        in_specs += [wspec, aspec, aspec, vspec, vspec, vspec]
        args += [W, a_s, a_d, b.reshape(1, -1), g.reshape(1, -1),
                 be.reshape(1, -1)]
    return pl.pallas_call(
        functools.partial(_fused_kernel, T=T, L=L, I=I, H=H),
        grid=(B,),
        in_specs=in_specs,
        out_specs=pl.BlockSpec((1, T, H), lambda b: (b, 0, 0)),
        out_shape=jax.ShapeDtypeStruct((B, T, H), jnp.float32),
        scratch_shapes=[pltpu.VMEM((N, N), jnp.float32),
                        pltpu.VMEM((N, N), jnp.float32)],
        interpret=interpret,
    )(*args)


def kernel(text_repr, label_repr, image_repr,
           W0, as0, ad0, b0, g0, be0,
           W1, as1, ad1, b1, g1, be1,
           W2, as2, ad2, b2, g2, be2):
    params = [
        (W0, as0, ad0, b0, g0, be0),
        (W1, as1, ad1, b1, g1, be1),
        (W2, as2, ad2, b2, g2, be2),
    ]
    return _run_fused(text_repr, label_repr, image_repr, params)
